# baseline (speedup 1.0000x reference)
import functools

import jax
import jax.numpy as jnp
from jax.experimental import pallas as pl
from jax.experimental.pallas import tpu as pltpu

D_IN = 768
D_SAE = 6144
T = 12
K = 32
B = 32
NQ = 4
QC = D_SAE // NQ
_NCH = D_SAE // 128
_TOPC = 6


def _topk_chain(v, n):
    def body(_, vv):
        m = jnp.max(vv, axis=1, keepdims=True)
        return jnp.where(vv >= m, -jnp.inf, vv)

    return jax.lax.fori_loop(0, n, body, v)


def _topk_threshold_fast(load_chunk):
    neg = jnp.full((B, 128), -jnp.inf, dtype=jnp.float32)
    M = [[neg] * _TOPC, [neg] * _TOPC]
    for s in range(_NCH // 2):
        for g in range(2):
            v = load_chunk(g * (_NCH // 2) + s)
            for i in range(_TOPC):
                hi = jnp.maximum(M[g][i], v)
                v = jnp.minimum(M[g][i], v)
                M[g][i] = hi
    thr = None
    for it in range(K):
        m = jnp.max(jnp.maximum(M[0][0], M[1][0]), axis=1, keepdims=True)
        thr = m
        if it < K - 1:
            for g in range(2):
                eq = M[g][0] == m
                for i in range(_TOPC - 1):
                    M[g][i] = jnp.where(eq, M[g][i + 1], M[g][i])
                M[g][_TOPC - 1] = jnp.where(eq, -jnp.inf, M[g][_TOPC - 1])
    return thr


def _fused_kernel(x_ref, b_dec_ref, b_enc_ref, w_enc_hbm, w_dec_ref,
                  u_ref, x_hat_ref, loss_ref,
                  pre_ref, thr_ref, acc_ref, w_buf, w_sems):
    t = pl.program_id(0)
    j = pl.program_id(1)
    enc_t = jnp.minimum(t, T - 1)
    dec_t = jnp.maximum(t - 1, 0)
    pb = jax.lax.rem(t, 2)
    qb = jax.lax.rem(t + 1, 2)

    def _w_copy(stack, q):
        slot = jax.lax.rem(stack, 2)
        return pltpu.make_async_copy(
            w_enc_hbm.at[stack, pl.ds(q * QC, QC), :],
            w_buf.at[slot, pl.ds(q * QC, QC), :],
            w_sems.at[slot, q],
        )

    @pl.when((t == 0) & (j == 0))
    def _first_fetch():
        for q in range(NQ):
            _w_copy(0, q).start()

    @pl.when((t < T - 1))
    def _next_fetch():
        for q in range(NQ):
            @pl.when(j == q)
            def _(q=q):
                _w_copy(t + 1, q).start()

    @pl.when((t < T) & (j == 0))
    def _enc():
        for q in range(NQ):
            _w_copy(enc_t, q).wait()
        xc = x_ref[:, enc_t, :] - b_dec_ref[enc_t]
        pre = jax.lax.dot_general(
            xc, w_buf[pb],
            dimension_numbers=(((1,), (1,)), ((), ())),
            preferred_element_type=jnp.float32,
        )
        pre = pre + b_enc_ref[enc_t]
        pre_ref[pb] = pre

    @pl.when((t < T) & (j == 1))
    def _thr():
        thr = _topk_threshold_fast(
            lambda c: pre_ref[pb, :, c * 128:(c + 1) * 128])
        thr_ref[pb] = jnp.broadcast_to(thr, (B, 128))

    @pl.when((t < T) & (j == 2))
    def _verify():
        thr = thr_ref[pb, :, 0:1]
        cnt = jnp.zeros((B, 128), dtype=jnp.float32)
        for c in range(_NCH):
            chunk = pre_ref[pb, :, c * 128:(c + 1) * 128]
            cnt = cnt + (chunk > thr).astype(jnp.float32)
        bad = jnp.max(jnp.sum(cnt, axis=1)) > (K - 1)

        @pl.when(bad)
        def _fallback():
            vals = _topk_chain(pre_ref[pb], K - 1)
            exact = jnp.max(vals, axis=1, keepdims=True)
            thr_ref[pb] = jnp.broadcast_to(exact, (B, 128))

    @pl.when(t > 0)
    def _dec():
        thr = thr_ref[qb, :, 0:1]
        chunk = pre_ref[qb, :, pl.ds(j * QC, QC)]
        u_c = jnp.where(chunk >= thr, jnp.maximum(chunk, 0.0), 0.0)
        u_ref[:, 0, 0, :] = u_c
        part = jax.lax.dot_general(
            u_c, w_dec_ref[0],
            dimension_numbers=(((1,), (1,)), ((), ())),
            preferred_element_type=jnp.float32,
        )

        @pl.when(j == 0)
        def _():
            acc_ref[...] = part

        @pl.when(j > 0)
        def _():
            acc_ref[...] += part

        @pl.when(j == NQ - 1)
        def _():
            xh = acc_ref[...] + b_dec_ref[dec_t]
            x_hat_ref[:, 0, 0, :] = xh
            d = xh - x_ref[:, dec_t, :]
            loss_ref[0, 0, 0] = jnp.sum(d * d)


@functools.partial(jax.jit, static_argnums=())
def kernel(x, b_dec, W_enc, b_enc, W_dec):
    b_dec3 = b_dec[:, None, :]
    b_enc3 = b_enc[:, None, :]

    u_t, x_hat_t, loss_acc = pl.pallas_call(
        _fused_kernel,
        grid=(T + 1, NQ),
        in_specs=[
            pl.BlockSpec((B, T, D_IN), lambda t, j: (0, 0, 0)),
            pl.BlockSpec((T, 1, D_IN), lambda t, j: (0, 0, 0)),
            pl.BlockSpec((T, 1, D_SAE), lambda t, j: (0, 0, 0)),
            pl.BlockSpec(memory_space=pltpu.HBM),
            pl.BlockSpec(
                (1, D_IN, QC),
                lambda t, j: (jnp.maximum(t - 1, 0), 0,
                              jnp.where(t > 0, j, 0))),
        ],
        out_specs=[
            pl.BlockSpec(
                (B, 1, 1, QC),
                lambda t, j: (0, jnp.maximum(t - 1, 0), 0,
                              jnp.where(t > 0, j, 0))),
            pl.BlockSpec((B, 1, 1, D_IN),
                         lambda t, j: (0, jnp.maximum(t - 1, 0), 0, 0)),
            pl.BlockSpec((1, 1, 1),
                         lambda t, j: (jnp.maximum(t - 1, 0), 0, 0),
                         memory_space=pltpu.SMEM),
        ],
        out_shape=[
            jax.ShapeDtypeStruct((B, T, 1, D_SAE), jnp.float32),
            jax.ShapeDtypeStruct((B, T, 1, D_IN), jnp.float32),
            jax.ShapeDtypeStruct((T, 1, 1), jnp.float32),
        ],
        scratch_shapes=[
            pltpu.VMEM((2, B, D_SAE), jnp.float32),
            pltpu.VMEM((2, B, 128), jnp.float32),
            pltpu.VMEM((B, D_IN), jnp.float32),
            pltpu.VMEM((2, D_SAE, D_IN), jnp.float32),
            pltpu.SemaphoreType.DMA((2, NQ)),
        ],
    )(x, b_dec3, b_enc3, W_enc, W_dec)

    recon_loss = jnp.sum(loss_acc) / (B * T)
    x_hat = x_hat_t.reshape(B, T, D_IN)
    u = u_t.reshape(B, T, D_SAE)
    return (recon_loss, x_hat, u)


# speedup vs baseline: 1.0987x; 1.0987x over previous
import functools

import jax
import jax.numpy as jnp
from jax.experimental import pallas as pl
from jax.experimental.pallas import tpu as pltpu

D_IN = 768
D_SAE = 6144
T = 12
K = 32
B = 32
NQ = 4
QC = D_SAE // NQ
RC = D_IN // NQ
_NCH = D_SAE // 128
_TOPC = 6


def _topk_chain(v, n):
    def body(_, vv):
        m = jnp.max(vv, axis=1, keepdims=True)
        return jnp.where(vv >= m, -jnp.inf, vv)

    return jax.lax.fori_loop(0, n, body, v)


def _topk_threshold_fast(load_chunk):
    neg = jnp.full((B, 128), -jnp.inf, dtype=jnp.float32)
    M = [[neg] * _TOPC, [neg] * _TOPC]
    for s in range(_NCH // 2):
        for g in range(2):
            v = load_chunk(g * (_NCH // 2) + s)
            for i in range(_TOPC):
                hi = jnp.maximum(M[g][i], v)
                v = jnp.minimum(M[g][i], v)
                M[g][i] = hi
    thr = None
    for it in range(K):
        m = jnp.max(jnp.maximum(M[0][0], M[1][0]), axis=1, keepdims=True)
        thr = m
        if it < K - 1:
            for g in range(2):
                eq = M[g][0] == m
                for i in range(_TOPC - 1):
                    M[g][i] = jnp.where(eq, M[g][i + 1], M[g][i])
                M[g][_TOPC - 1] = jnp.where(eq, -jnp.inf, M[g][_TOPC - 1])
    return thr


def _fused_kernel(x_ref, b_dec_ref, b_enc_ref, w_enc_hbm, w_dec_ref,
                  u_ref, x_hat_ref, loss_ref,
                  pre_ref, thr_ref, acc_ref, u_scr_ref, w_buf, w_sems):
    t = pl.program_id(0)
    j = pl.program_id(1)
    enc_t = jnp.minimum(t, T - 1)
    dec_t = jnp.maximum(t - 1, 0)
    pb = jax.lax.rem(t, 2)
    qb = jax.lax.rem(t + 1, 2)

    def _w_copy(stack, q):
        slot = jax.lax.rem(stack, 2)
        return pltpu.make_async_copy(
            w_enc_hbm.at[stack, pl.ds(q * QC, QC), :],
            w_buf.at[slot, pl.ds(q * QC, QC), :],
            w_sems.at[slot, q],
        )

    @pl.when((t == 0) & (j == 0))
    def _first_fetch():
        for q in range(NQ):
            _w_copy(0, q).start()

    @pl.when((t < T - 1))
    def _next_fetch():
        for q in range(NQ):
            @pl.when(j == q)
            def _(q=q):
                _w_copy(t + 1, q).start()

    @pl.when((t < T) & (j == 0))
    def _enc():
        for q in range(NQ):
            _w_copy(enc_t, q).wait()
        xc = x_ref[enc_t] - b_dec_ref[enc_t]
        pre = jax.lax.dot_general(
            xc, w_buf[pb],
            dimension_numbers=(((1,), (1,)), ((), ())),
            preferred_element_type=jnp.float32,
        )
        pre = pre + b_enc_ref[enc_t]
        pre_ref[pb] = pre

    @pl.when((t < T) & (j == 1))
    def _thr():
        thr = _topk_threshold_fast(
            lambda c: pre_ref[pb, :, c * 128:(c + 1) * 128])
        thr_ref[pb] = jnp.broadcast_to(thr, (B, 128))

    @pl.when((t < T) & (j == 2))
    def _verify():
        thr = thr_ref[pb, :, 0:1]
        cnt = jnp.zeros((B, 128), dtype=jnp.float32)
        for c in range(_NCH):
            chunk = pre_ref[pb, :, c * 128:(c + 1) * 128]
            cnt = cnt + (chunk > thr).astype(jnp.float32)
        bad = jnp.max(jnp.sum(cnt, axis=1)) > (K - 1)

        @pl.when(bad)
        def _fallback():
            vals = _topk_chain(pre_ref[pb], K - 1)
            exact = jnp.max(vals, axis=1, keepdims=True)
            thr_ref[pb] = jnp.broadcast_to(exact, (B, 128))

    @pl.when((t < T) & (j == 3))
    def _build_u():
        thr = thr_ref[pb, :, 0:1]
        p = pre_ref[pb]
        u_scr_ref[pb] = jnp.where(p >= thr, jnp.maximum(p, 0.0), 0.0)

    @pl.when(t > 0)
    def _dec():
        u_ref[0] = u_scr_ref[qb, :, pl.ds(j * QC, QC)]
        part = jax.lax.dot_general(
            u_scr_ref[qb], w_dec_ref[0],
            dimension_numbers=(((1,), (1,)), ((), ())),
            preferred_element_type=jnp.float32,
        )

        @pl.when(j < NQ - 1)
        def _():
            acc_ref[j] = part

        @pl.when(j == NQ - 1)
        def _():
            xh = jnp.concatenate(
                [acc_ref[0], acc_ref[1], acc_ref[2], part], axis=1)
            xh = xh + b_dec_ref[dec_t]
            x_hat_ref[0] = xh
            d = xh - x_ref[dec_t]
            loss_ref[0, 0, 0] = jnp.sum(d * d)


@functools.partial(jax.jit, static_argnums=())
def kernel(x, b_dec, W_enc, b_enc, W_dec):
    x_t = jnp.transpose(x, (1, 0, 2))
    b_dec3 = b_dec[:, None, :]
    b_enc3 = b_enc[:, None, :]

    u_t, x_hat_t, loss_acc = pl.pallas_call(
        _fused_kernel,
        grid=(T + 1, NQ),
        in_specs=[
            pl.BlockSpec((T, B, D_IN), lambda t, j: (0, 0, 0)),
            pl.BlockSpec((T, 1, D_IN), lambda t, j: (0, 0, 0)),
            pl.BlockSpec((T, 1, D_SAE), lambda t, j: (0, 0, 0)),
            pl.BlockSpec(memory_space=pltpu.HBM),
            pl.BlockSpec(
                (1, RC, D_SAE),
                lambda t, j: (jnp.maximum(t - 1, 0),
                              jnp.where(t > 0, j, 0), 0)),
        ],
        out_specs=[
            pl.BlockSpec(
                (1, B, QC),
                lambda t, j: (jnp.maximum(t - 1, 0), 0,
                              jnp.where(t > 0, j, 0))),
            pl.BlockSpec((1, B, D_IN),
                         lambda t, j: (jnp.maximum(t - 1, 0), 0, 0)),
            pl.BlockSpec((1, 1, 1),
                         lambda t, j: (jnp.maximum(t - 1, 0), 0, 0),
                         memory_space=pltpu.SMEM),
        ],
        out_shape=[
            jax.ShapeDtypeStruct((T, B, D_SAE), jnp.float32),
            jax.ShapeDtypeStruct((T, B, D_IN), jnp.float32),
            jax.ShapeDtypeStruct((T, 1, 1), jnp.float32),
        ],
        scratch_shapes=[
            pltpu.VMEM((2, B, D_SAE), jnp.float32),
            pltpu.VMEM((2, B, 128), jnp.float32),
            pltpu.VMEM((NQ - 1, B, RC), jnp.float32),
            pltpu.VMEM((2, B, D_SAE), jnp.float32),
            pltpu.VMEM((2, D_SAE, D_IN), jnp.float32),
            pltpu.SemaphoreType.DMA((2, NQ)),
        ],
    )(x_t, b_dec3, b_enc3, W_enc, W_dec)

    recon_loss = jnp.sum(loss_acc) / (B * T)
    x_hat = jnp.transpose(x_hat_t, (1, 0, 2))
    u = jnp.transpose(u_t, (1, 0, 2))
    return (recon_loss, x_hat, u)
